# baseline (speedup 1.0000x reference)
import functools

import jax
import jax.numpy as jnp
from jax import lax
from jax.experimental import pallas as pl
from jax.experimental.pallas import tpu as pltpu
from jax.experimental.pallas import tpu_sc as plsc

N = 10000
E = 320000
D = 128
NC = 2
NS = 16
NW = NC * NS
K = 128
NB = 80
EPT = NB * K
EPAD = NW * EPT
IR = 4
NPAD = 10112
RPT = NPAD // NS
CW = 16


def _sc_aggregate(x, src3, dst3, with_counts):
    mesh = plsc.VectorSubcoreMesh(core_axis_name="c", subcore_axis_name="s")

    @functools.partial(
        pl.kernel,
        out_type=[
            jax.ShapeDtypeStruct((NC, NPAD, D), jnp.float32),
            jax.ShapeDtypeStruct((NC, NPAD, CW), jnp.float32),
        ],
        mesh=mesh,
        compiler_params=pltpu.CompilerParams(use_tc_tiling_on_sc=False),
        scratch_types=[
            [pltpu.VMEM((K,), jnp.int32) for _ in range(IR)],
            [pltpu.VMEM((K,), jnp.int32) for _ in range(IR)],
            [pltpu.VMEM((K, D), jnp.float32) for _ in range(2)],
            pltpu.VMEM((K, CW), jnp.float32),
            pltpu.VMEM((K, CW), jnp.float32),
            pltpu.VMEM_SHARED((NPAD, D), jnp.float32),
            pltpu.VMEM_SHARED((NPAD, CW), jnp.float32),
            [pltpu.SemaphoreType.DMA for _ in range(IR)],
            [pltpu.SemaphoreType.DMA for _ in range(2)],
            [pltpu.SemaphoreType.DMA for _ in range(2)],
            [pltpu.SemaphoreType.DMA for _ in range(2)],
        ],
    )
    def agg_kernel(x_hbm, src_hbm, dst_hbm, out_hbm, outc_hbm,
                   srcb, dstb, rows, ones, zcnt, acc_s, cnt_s,
                   isem, gs, ss, cs):
        c = lax.axis_index("c")
        s = lax.axis_index("s")
        wid = s * NC + c

        def fill_zrow(i, _):
            for j in range(D // 16):
                rows[0][i, pl.ds(j * 16, 16)] = jnp.zeros((16,),
                                                          jnp.float32)
            return 0
        lax.fori_loop(0, K, fill_zrow, 0)

        def fill_zcnt(i, _):
            zcnt[i, :] = jnp.zeros((16,), jnp.float32)
            if with_counts:
                ones[i, :] = jnp.ones((16,), jnp.float32)
            return 0
        lax.fori_loop(0, K, fill_zcnt, 0)

        base = s * RPT
        tail = RPT - 4 * K
        for z in range(4):
            pltpu.sync_copy(rows[0], acc_s.at[pl.ds(base + z * K, K)])
        pltpu.sync_copy(rows[0].at[pl.ds(0, tail)],
                        acc_s.at[pl.ds(base + 4 * K, tail)])
        if with_counts:
            for z in range(4):
                pltpu.sync_copy(zcnt, cnt_s.at[pl.ds(base + z * K, K)])
            pltpu.sync_copy(zcnt.at[pl.ds(0, tail)],
                            cnt_s.at[pl.ds(base + 4 * K, tail)])

        plsc.subcore_barrier()

        def idx_start(i, r):
            pltpu.async_copy(src_hbm.at[wid, i], srcb[r], isem[r])
            pltpu.async_copy(dst_hbm.at[wid, i], dstb[r], isem[r])

        def idx_wait(r):
            pltpu.make_async_copy(src_hbm.at[wid, 0], srcb[r],
                                  isem[r]).wait()
            pltpu.make_async_copy(dst_hbm.at[wid, 0], dstb[r],
                                  isem[r]).wait()

        def g_start(r, b):
            pltpu.async_copy(x_hbm.at[srcb[r]], rows[b], gs[b])

        def g_wait(b):
            pltpu.make_async_copy(x_hbm.at[srcb[0]], rows[b], gs[b]).wait()

        def s_start(r, b):
            pltpu.async_copy(rows[b], acc_s.at[dstb[r]], ss[b], add=True)

        def s_wait(b):
            pltpu.make_async_copy(rows[b], acc_s.at[dstb[0]], ss[b]).wait()

        def c_start(r, b):
            pltpu.async_copy(ones, cnt_s.at[dstb[r]], cs[b], add=True)

        def c_wait(b):
            pltpu.make_async_copy(ones, cnt_s.at[dstb[0]], cs[b]).wait()

        for r in range(IR - 1):
            idx_start(r, r)
        idx_wait(0)
        g_start(0, 0)

        NSW4 = NB // 4

        def body4(j, _):
            for b in range(4):
                bb = b % 2
                g_wait(bb)
                s_start(b, bb)
                if with_counts:
                    c_start(b, bb)

                def drain():
                    s_wait(1 - bb)
                    if with_counts:
                        c_wait(1 - bb)
                if b == 0:
                    @pl.when(j > 0)
                    def _():
                        drain()
                else:
                    drain()

                rpre = (b + 3) % IR
                if b == 0:
                    idx_start(4 * j + 3, rpre)
                else:
                    @pl.when(j < NSW4 - 1)
                    def _():
                        idx_start(4 * j + b + 3, rpre)

                def nxt():
                    idx_wait((b + 1) % IR)
                    g_start((b + 1) % IR, 1 - bb)
                if b < 3:
                    nxt()
                else:
                    @pl.when(j < NSW4 - 1)
                    def _():
                        nxt()
            return 0
        lax.fori_loop(0, NSW4, body4, 0)

        s_wait(1)
        if with_counts:
            c_wait(1)

        plsc.subcore_barrier()
        pltpu.sync_copy(acc_s.at[pl.ds(base, RPT)],
                        out_hbm.at[c, pl.ds(base, RPT)])
        if with_counts:
            pltpu.sync_copy(cnt_s.at[pl.ds(base, RPT)],
                            outc_hbm.at[c, pl.ds(base, RPT)])

    return agg_kernel(x, src3, dst3)


def _tc_layer1(S, C, x, Wl1, bl1, Wr1, gamma, beta, Wr2, bl2):
    def body(S_ref, C_ref, x_ref, Wl1_ref, bl1_ref, Wr1_ref, g_ref, b_ref,
             Wr2_ref, bl2_ref, h2_ref, r2_ref):
        cnt = jnp.maximum(C_ref[0, :N, 0:1] + C_ref[1, :N, 0:1], 1.0)
        agg = (S_ref[0, :N, :] + S_ref[1, :N, :]) / cnt
        h = (jnp.dot(agg, Wl1_ref[...], preferred_element_type=jnp.float32)
             + jnp.dot(x_ref[...], Wr1_ref[...],
                       preferred_element_type=jnp.float32)
             + bl1_ref[...])
        mu = jnp.mean(h, axis=0, keepdims=True)
        var = jnp.mean((h - mu) * (h - mu), axis=0, keepdims=True)
        hn = (h - mu) / jnp.sqrt(var + 1e-5) * g_ref[...] + b_ref[...]
        h2 = jnp.where(hn >= 0, hn, 0.01 * hn)
        h2_ref[...] = h2
        r2_ref[...] = (jnp.dot(h2, Wr2_ref[...],
                               preferred_element_type=jnp.float32)
                       + bl2_ref[...])

    return pl.pallas_call(
        body,
        out_shape=[
            jax.ShapeDtypeStruct((N, D), jnp.float32),
            jax.ShapeDtypeStruct((N, D), jnp.float32),
        ],
    )(S, C, x, Wl1, bl1, Wr1, gamma, beta, Wr2, bl2)


def _tc_layer2(S2, C, r2, Wl2):
    def body(S_ref, C_ref, r2_ref, Wl2_ref, out_ref):
        cnt = jnp.maximum(C_ref[0, :N, 0:1] + C_ref[1, :N, 0:1], 1.0)
        agg = (S_ref[0, :N, :] + S_ref[1, :N, :]) / cnt
        out_ref[...] = (jnp.dot(agg, Wl2_ref[...],
                                preferred_element_type=jnp.float32)
                        + r2_ref[...])

    return pl.pallas_call(
        body,
        out_shape=jax.ShapeDtypeStruct((N, D), jnp.float32),
    )(S2, C, r2, Wl2)


def kernel(x, edge_index, Wl1, bl1, Wr1, gamma, beta, Wl2, bl2, Wr2):
    pad_src = jnp.zeros((EPAD - E,), jnp.int32)
    pad_dst = jnp.full((EPAD - E,), N, jnp.int32)
    src3 = jnp.concatenate(
        [edge_index[0].astype(jnp.int32), pad_src]).reshape(NW, NB, K)
    dst3 = jnp.concatenate(
        [edge_index[1].astype(jnp.int32), pad_dst]).reshape(NW, NB, K)
    bl1r = bl1.reshape(1, D)
    bl2r = bl2.reshape(1, D)
    gr = gamma.reshape(1, D)
    br = beta.reshape(1, D)

    S1, C = _sc_aggregate(x, src3, dst3, with_counts=True)
    h2, r2 = _tc_layer1(S1, C, x, Wl1, bl1r, Wr1, gr, br, Wr2, bl2r)
    S2, _ = _sc_aggregate(h2, src3, dst3, with_counts=False)
    return _tc_layer2(S2, C, r2, Wl2)


# speedup vs baseline: 1.8478x; 1.8478x over previous
import functools

import jax
import jax.numpy as jnp
from jax import lax
from jax.experimental import pallas as pl
from jax.experimental.pallas import tpu as pltpu
from jax.experimental.pallas import tpu_sc as plsc

N = 10000
E = 320000
D = 128
HD = D // 2
NC = 2
NS = 16
K = 80
NB = 250
EPT = NB * K
NPAD = 10112
RPT = NPAD // NS
CW = 16


def _sc_aggregate(x2, src3, src3p, dst3, with_counts):
    mesh = plsc.VectorSubcoreMesh(core_axis_name="c", subcore_axis_name="s")

    @functools.partial(
        pl.kernel,
        out_type=[
            jax.ShapeDtypeStruct((NC, NPAD, HD), jnp.float32),
            jax.ShapeDtypeStruct((NPAD, CW), jnp.float32),
        ],
        mesh=mesh,
        compiler_params=pltpu.CompilerParams(use_tc_tiling_on_sc=False),
        scratch_types=[
            pltpu.VMEM((NB, K), jnp.int32),
            pltpu.VMEM((NB, K), jnp.int32),
            [pltpu.VMEM((K, HD), jnp.float32) for _ in range(2)],
            pltpu.VMEM((K, CW), jnp.float32),
            pltpu.VMEM((K, CW), jnp.float32),
            pltpu.VMEM_SHARED((NPAD, HD), jnp.float32),
            pltpu.VMEM_SHARED((NPAD, CW), jnp.float32),
            [pltpu.SemaphoreType.DMA for _ in range(2)],
            [pltpu.SemaphoreType.DMA for _ in range(2)],
            [pltpu.SemaphoreType.DMA for _ in range(2)],
        ],
    )
    def agg_kernel(x_hbm, src_hbm, srcp_hbm, dst_hbm, out_hbm, outc_hbm,
                   srcv, dstv, rows, ones, zcnt, acc_s, cnt_s, gs, ss, cs):
        c = lax.axis_index("c")
        s = lax.axis_index("s")

        def fill_zrow(i, _):
            for j in range(HD // 16):
                rows[0][i, pl.ds(j * 16, 16)] = jnp.zeros((16,),
                                                          jnp.float32)
            return 0
        lax.fori_loop(0, K, fill_zrow, 0)

        def fill_zcnt(i, _):
            zcnt[i, :] = jnp.zeros((16,), jnp.float32)
            if with_counts:
                ones[i, :] = jnp.ones((16,), jnp.float32)
            return 0
        lax.fori_loop(0, K, fill_zcnt, 0)

        base = s * RPT
        nz = RPT // K
        tail = RPT - nz * K
        for z in range(nz):
            pltpu.sync_copy(rows[0], acc_s.at[pl.ds(base + z * K, K)])
        pltpu.sync_copy(rows[0].at[pl.ds(0, tail)],
                        acc_s.at[pl.ds(base + nz * K, tail)])
        if with_counts:
            @pl.when(c == 0)
            def _():
                for z in range(nz):
                    pltpu.sync_copy(zcnt, cnt_s.at[pl.ds(base + z * K, K)])
                pltpu.sync_copy(zcnt.at[pl.ds(0, tail)],
                                cnt_s.at[pl.ds(base + nz * K, tail)])

        @pl.when(c == 0)
        def _():
            pltpu.sync_copy(src_hbm.at[s], srcv)

        @pl.when(c == 1)
        def _():
            pltpu.sync_copy(srcp_hbm.at[s], srcv)

        pltpu.sync_copy(dst_hbm.at[s], dstv)

        plsc.subcore_barrier()

        def g_start(i, b):
            pltpu.async_copy(x_hbm.at[srcv.at[i]], rows[b], gs[b])

        def g_wait(b):
            pltpu.make_async_copy(x_hbm.at[srcv.at[0]], rows[b],
                                  gs[b]).wait()

        def s_start(i, b):
            pltpu.async_copy(rows[b], acc_s.at[dstv.at[i]], ss[b], add=True)

        def s_wait(b):
            pltpu.make_async_copy(rows[b], acc_s.at[dstv.at[0]],
                                  ss[b]).wait()

        def cnt_fire(j, i, b):
            @pl.when(c == 0)
            def _():
                @pl.when(j > 0)
                def _():
                    pltpu.make_async_copy(ones, cnt_s.at[dstv.at[0]],
                                          cs[b]).wait()
                pltpu.async_copy(ones, cnt_s.at[dstv.at[i]], cs[b],
                                 add=True)

        g_start(0, 0)

        def body(j, _):
            i0 = 2 * j
            i1 = i0 + 1
            g_wait(0)
            s_start(i0, 0)

            @pl.when(j > 0)
            def _():
                s_wait(1)
            g_start(i1, 1)
            if with_counts:
                cnt_fire(j, i0, 0)

            g_wait(1)
            s_start(i1, 1)
            s_wait(0)

            @pl.when(j < NB // 2 - 1)
            def _():
                g_start(i0 + 2, 0)
            if with_counts:
                cnt_fire(j, i1, 1)
            return 0
        lax.fori_loop(0, NB // 2, body, 0)

        s_wait(1)
        if with_counts:
            @pl.when(c == 0)
            def _():
                pltpu.make_async_copy(ones, cnt_s.at[dstv.at[0]],
                                      cs[0]).wait()
                pltpu.make_async_copy(ones, cnt_s.at[dstv.at[0]],
                                      cs[1]).wait()

        plsc.subcore_barrier()
        pltpu.sync_copy(acc_s.at[pl.ds(base, RPT)],
                        out_hbm.at[c, pl.ds(base, RPT)])
        if with_counts:
            @pl.when(c == 0)
            def _():
                pltpu.sync_copy(cnt_s.at[pl.ds(base, RPT)],
                                outc_hbm.at[pl.ds(base, RPT)])

    return agg_kernel(x2, src3, src3p, dst3)


def _split_stack(h):
    return jnp.concatenate([h[:, :HD], h[:, HD:]], axis=0)


def _tc_layer1(S, C, x, Wl1, bl1, Wr1, gamma, beta, Wr2, bl2):
    def body(S_ref, C_ref, x_ref, Wl1_ref, bl1_ref, Wr1_ref, g_ref, b_ref,
             Wr2_ref, bl2_ref, h2_ref, r2_ref):
        inv = 1.0 / jnp.maximum(C_ref[:N, 0:1], 1.0)
        aggL = S_ref[0, :N, :] * inv
        aggR = S_ref[1, :N, :] * inv
        h = (jnp.dot(aggL, Wl1_ref[:HD, :],
                     preferred_element_type=jnp.float32)
             + jnp.dot(aggR, Wl1_ref[HD:, :],
                       preferred_element_type=jnp.float32)
             + jnp.dot(x_ref[...], Wr1_ref[...],
                       preferred_element_type=jnp.float32)
             + bl1_ref[...])
        mu = jnp.mean(h, axis=0, keepdims=True)
        var = jnp.mean((h - mu) * (h - mu), axis=0, keepdims=True)
        hn = (h - mu) / jnp.sqrt(var + 1e-5) * g_ref[...] + b_ref[...]
        h2 = jnp.where(hn >= 0, hn, 0.01 * hn)
        h2_ref[...] = h2
        r2_ref[...] = (jnp.dot(h2, Wr2_ref[...],
                               preferred_element_type=jnp.float32)
                       + bl2_ref[...])

    return pl.pallas_call(
        body,
        out_shape=[
            jax.ShapeDtypeStruct((N, D), jnp.float32),
            jax.ShapeDtypeStruct((N, D), jnp.float32),
        ],
    )(S, C, x, Wl1, bl1, Wr1, gamma, beta, Wr2, bl2)


def _tc_layer2(S2, C, r2, Wl2):
    def body(S_ref, C_ref, r2_ref, Wl2_ref, out_ref):
        inv = 1.0 / jnp.maximum(C_ref[:N, 0:1], 1.0)
        aggL = S_ref[0, :N, :] * inv
        aggR = S_ref[1, :N, :] * inv
        out_ref[...] = (jnp.dot(aggL, Wl2_ref[:HD, :],
                                preferred_element_type=jnp.float32)
                        + jnp.dot(aggR, Wl2_ref[HD:, :],
                                  preferred_element_type=jnp.float32)
                        + r2_ref[...])

    return pl.pallas_call(
        body,
        out_shape=jax.ShapeDtypeStruct((N, D), jnp.float32),
    )(S2, C, r2, Wl2)


def kernel(x, edge_index, Wl1, bl1, Wr1, gamma, beta, Wl2, bl2, Wr2):
    src3 = edge_index[0].astype(jnp.int32).reshape(NS, NB, K)
    src3p = src3 + N
    dst3 = edge_index[1].astype(jnp.int32).reshape(NS, NB, K)
    bl1r = bl1.reshape(1, D)
    bl2r = bl2.reshape(1, D)
    gr = gamma.reshape(1, D)
    br = beta.reshape(1, D)

    S1, C = _sc_aggregate(_split_stack(x), src3, src3p, dst3,
                          with_counts=True)
    h2, r2 = _tc_layer1(S1, C, x, Wl1, bl1r, Wr1, gr, br, Wr2, bl2r)
    S2, _ = _sc_aggregate(_split_stack(h2), src3, src3p, dst3,
                          with_counts=False)
    return _tc_layer2(S2, C, r2, Wl2)
